# baseline (speedup 1.0000x reference)
import math

import jax
import jax.numpy as jnp
from jax import lax
from jax.experimental import pallas as pl
from jax.experimental.pallas import tpu as pltpu
from jax.experimental.pallas import tpu_sc as plsc

N = 10000
E = 320000
D = 128
H = 8
DH = D // H
C = 40
HOPS = 4
ALPHA = 0.15
MAX_DEG = 128

NS = 16
L = 16
N_PAD = 10240
NPT = N_PAD // NS
SUB = 80
EPT1 = E // NS
EB = 2000
NB = EPT1 // EB
NW = 2 * NS
EPT2 = E // NW
B2G = 80
NBG = EPT2 // B2G
B2 = 80
NB2 = EPT2 // B2
BZ = 400
NBZ = EPT1 // BZ
ZPT = N_PAD * L // NS

_mesh1 = plsc.VectorSubcoreMesh(
    core_axis_name="c", subcore_axis_name="s", num_cores=1)
_mesh2 = plsc.VectorSubcoreMesh(
    core_axis_name="c", subcore_axis_name="s", num_cores=2)


def _deg_body(dst_hbm, emb_hbm, xw_hbm, h_hbm,
              cnt_sh, onesb, dstbuf, mycnt, degidx, embbuf, xwbuf, hbuf):
    s = lax.axis_index("s")
    zeros = jnp.zeros((L,), jnp.float32)
    ones = jnp.full((L,), 1.0, jnp.float32)

    def zb_body(i, _):
        mycnt[pl.ds(i * L, L)] = zeros
        return 0
    lax.fori_loop(0, NPT // L, zb_body, 0)
    pltpu.sync_copy(mycnt, cnt_sh.at[pl.ds(s * NPT, NPT)])

    def ob_body(i, _):
        onesb[pl.ds(i * L, L)] = ones
        return 0
    lax.fori_loop(0, EB // L, ob_body, 0)
    plsc.subcore_barrier()

    for b in range(NB):
        pltpu.sync_copy(dst_hbm.at[pl.ds(s * EPT1 + b * EB, EB)], dstbuf)
        pltpu.sync_copy(onesb, cnt_sh.at[dstbuf], add=True)
    plsc.subcore_barrier()

    pltpu.sync_copy(cnt_sh.at[pl.ds(s * NPT, NPT)], mycnt)

    def cl_body(i, _):
        degidx[i // 5, pl.ds((i % 5) * L, L)] = jnp.minimum(
            mycnt[pl.ds(i * L, L)], float(MAX_DEG - 1)).astype(jnp.int32)
        return 0
    lax.fori_loop(0, NPT // L, cl_body, 0)

    for c in range(NPT // SUB):
        node_base = s * NPT + c * SUB

        @pl.when(node_base < N)
        def _():
            pltpu.sync_copy(emb_hbm.at[degidx.at[c]], embbuf)
            pltpu.sync_copy(xw_hbm.at[pl.ds(node_base, SUB)], xwbuf)

            def add_body(i, _):
                r = i // 8
                j = (i % 8) * L
                hbuf[r, pl.ds(j, L)] = (xwbuf[r, pl.ds(j, L)]
                                        + embbuf[r, pl.ds(j, L)])
                return 0
            lax.fori_loop(0, SUB * 8, add_body, 0)
            pltpu.sync_copy(hbuf, h_hbm.at[pl.ds(node_base, SUB)])


def _deg_embed(dst, deg_emb, xw):
    return pl.kernel(
        _deg_body,
        out_type=jax.ShapeDtypeStruct((N, D), jnp.float32),
        mesh=_mesh1,
        scratch_types=[
            pltpu.VMEM_SHARED((N_PAD,), jnp.float32),
            pltpu.VMEM((EB,), jnp.float32),
            pltpu.VMEM((EB,), jnp.int32),
            pltpu.VMEM((NPT,), jnp.float32),
            pltpu.VMEM((NPT // SUB, SUB), jnp.int32),
            pltpu.VMEM((SUB, D), jnp.float32),
            pltpu.VMEM((SUB, D), jnp.float32),
            pltpu.VMEM((SUB, D), jnp.float32),
        ],
    )(dst, deg_emb, xw)


def _gather_body(src_hbm, dst_hbm, q_hbm, k_hbm, qd_hbm, ks_hbm,
                 srcb0, srcb1, dstb0, dstb1, qd0, qd1, ks0, ks1,
                 gsem0, gsem1, wsem0, wsem1):
    c = lax.axis_index("c")
    s = lax.axis_index("s")
    w = s * 2 + c
    buf0 = (srcb0, dstb0, qd0, ks0, gsem0, wsem0)
    buf1 = (srcb1, dstb1, qd1, ks1, gsem1, wsem1)

    def step(b, buf):
        srcb, dstb, qd, ks, gsem, wsem = buf
        base = w * EPT2 + b * B2G
        pltpu.sync_copy(dst_hbm.at[pl.ds(base, B2G)], dstb)
        pltpu.sync_copy(src_hbm.at[pl.ds(base, B2G)], srcb)

        @pl.when(b >= 2)
        def _():
            pltpu.make_async_copy(qd, qd_hbm.at[pl.ds(base, B2G)],
                                  wsem).wait()
            pltpu.make_async_copy(ks, ks_hbm.at[pl.ds(base, B2G)],
                                  wsem).wait()
        pltpu.async_copy(q_hbm.at[dstb], qd, gsem)
        pltpu.async_copy(k_hbm.at[srcb], ks, gsem)
        pltpu.make_async_copy(q_hbm.at[dstb], qd, gsem).wait()
        pltpu.make_async_copy(k_hbm.at[srcb], ks, gsem).wait()
        pltpu.async_copy(qd, qd_hbm.at[pl.ds(base, B2G)], wsem)
        pltpu.async_copy(ks, ks_hbm.at[pl.ds(base, B2G)], wsem)

    def batch_body(b, _):
        even = (b % 2) == 0

        @pl.when(even)
        def _():
            step(b, buf0)

        @pl.when(jnp.logical_not(even))
        def _():
            step(b, buf1)
        return 0
    lax.fori_loop(0, NBG, batch_body, 0)
    for buf in (buf0, buf1):
        srcb, dstb, qd, ks, gsem, wsem = buf
        pltpu.make_async_copy(qd, qd_hbm.at[pl.ds(0, B2G)], wsem).wait()
        pltpu.make_async_copy(ks, ks_hbm.at[pl.ds(0, B2G)], wsem).wait()


def _gather_qk(src, dst, q, k):
    return pl.kernel(
        _gather_body,
        out_type=(jax.ShapeDtypeStruct((E, D), jnp.float32),
                  jax.ShapeDtypeStruct((E, D), jnp.float32)),
        mesh=_mesh2,
        scratch_types=[
            pltpu.VMEM((B2G,), jnp.int32),
            pltpu.VMEM((B2G,), jnp.int32),
            pltpu.VMEM((B2G,), jnp.int32),
            pltpu.VMEM((B2G,), jnp.int32),
            pltpu.VMEM((B2G, D), jnp.float32),
            pltpu.VMEM((B2G, D), jnp.float32),
            pltpu.VMEM((B2G, D), jnp.float32),
            pltpu.VMEM((B2G, D), jnp.float32),
            pltpu.SemaphoreType.DMA,
            pltpu.SemaphoreType.DMA,
            pltpu.SemaphoreType.DMA,
            pltpu.SemaphoreType.DMA,
        ],
    )(src, dst, q, k)


def _edge_body(qd_ref, ks_ref, o_ref):
    blk = qd_ref.shape[0]
    prod = qd_ref[...] * ks_ref[...]
    row = lax.broadcasted_iota(jnp.int32, (D, L), 0) // DH
    col = lax.broadcasted_iota(jnp.int32, (D, L), 1)
    sel = jnp.where(row == col, 1.0, 0.0).astype(jnp.float32)
    e16 = jnp.dot(prod, sel, preferred_element_type=jnp.float32)
    mask = (lax.broadcasted_iota(jnp.int32, (blk, L), 1) < H).astype(
        jnp.float32)
    o_ref[...] = jnp.exp(e16 * (1.0 / math.sqrt(DH))) * mask


def _edge_ex(qd, ks):
    blk = 2000
    spec = pl.BlockSpec((blk, D), lambda i: (i, 0))
    return pl.pallas_call(
        _edge_body,
        grid=(E // blk,),
        in_specs=[spec, spec],
        out_specs=pl.BlockSpec((blk, L), lambda i: (i, 0)),
        out_shape=jax.ShapeDtypeStruct((E, L), jnp.float32),
    )(qd, ks)


def _z_body(dst_hbm, exf_hbm, zinv_hbm,
            z_sh, dstb, zsrc, zidx, zbuf):
    s = lax.axis_index("s")
    iota = lax.iota(jnp.int32, L)
    zeros = jnp.zeros((L,), jnp.float32)

    def zz_body(i, _):
        zbuf[pl.ds(i * L, L)] = zeros
        return 0
    lax.fori_loop(0, ZPT // L, zz_body, 0)
    pltpu.sync_copy(zbuf, z_sh.at[pl.ds(s * ZPT, ZPT)])
    plsc.subcore_barrier()

    for b in range(NBZ):
        base = s * EPT1 + b * BZ
        pltpu.sync_copy(dst_hbm.at[pl.ds(base, BZ)], dstb)
        pltpu.sync_copy(exf_hbm.at[pl.ds(base * L, BZ * L)], zsrc)

        def grp_body(g, _):
            dv = dstb[pl.ds(g * L, L)]
            for j in range(L):
                zidx[pl.ds((g * L + j) * L, L)] = dv[j] * L + iota
            return 0
        lax.fori_loop(0, BZ // L, grp_body, 0)
        pltpu.sync_copy(zsrc, z_sh.at[zidx], add=True)
    plsc.subcore_barrier()

    pltpu.sync_copy(z_sh.at[pl.ds(s * ZPT, ZPT)], zbuf)

    def zi_body(i, _):
        zbuf[pl.ds(i * L, L)] = 1.0 / (zbuf[pl.ds(i * L, L)] + 1e-9)
        return 0
    lax.fori_loop(0, ZPT // L, zi_body, 0)
    pltpu.sync_copy(zbuf, zinv_hbm.at[pl.ds(s * ZPT, ZPT)])


def _z_inv(dst, exf):
    return pl.kernel(
        _z_body,
        out_type=jax.ShapeDtypeStruct((N_PAD * L,), jnp.float32),
        mesh=_mesh1,
        scratch_types=[
            pltpu.VMEM_SHARED((N_PAD * L,), jnp.float32),
            pltpu.VMEM((BZ,), jnp.int32),
            pltpu.VMEM((BZ * L,), jnp.float32),
            pltpu.VMEM((BZ * L,), jnp.int32),
            pltpu.VMEM((ZPT,), jnp.float32),
        ],
    )(dst, exf)


def _hop_body(src_hbm, dst_hbm, exf_hbm, feat_hbm, pp_hbm,
              agg_sh, srcb, dstb0, dstb1, exb0, exb1, fbuf0, fbuf1,
              gsem0, gsem1, ssem0, ssem1):
    c = lax.axis_index("c")
    s = lax.axis_index("s")
    w = s * 2 + c
    zeros = jnp.zeros((L,), jnp.float32)
    buf0 = (dstb0, exb0, fbuf0, gsem0, ssem0)
    buf1 = (dstb1, exb1, fbuf1, gsem1, ssem1)

    def zm_body(i, _):
        fbuf0[i // 8, pl.ds((i % 8) * L, L)] = zeros
        return 0
    lax.fori_loop(0, B2 * 8, zm_body, 0)
    for r in range(NPT // B2):
        pltpu.sync_copy(fbuf0, agg_sh.at[pl.ds(s * NPT + r * B2, B2)])
    plsc.subcore_barrier()

    def issue(bi, buf):
        dstb, exb, fbuf, gsem, _ = buf
        base = w * EPT2 + bi * B2
        pltpu.sync_copy(src_hbm.at[pl.ds(base, B2)], srcb)
        pltpu.sync_copy(dst_hbm.at[pl.ds(base, B2)], dstb)
        pltpu.async_copy(feat_hbm.at[srcb], fbuf, gsem)
        pltpu.async_copy(exf_hbm.at[pl.ds(base * L, B2 * L)], exb, gsem)

    def wait_gather(bi, buf):
        dstb, exb, fbuf, gsem, _ = buf
        base = w * EPT2 + bi * B2
        pltpu.make_async_copy(feat_hbm.at[srcb], fbuf, gsem).wait()
        pltpu.make_async_copy(
            exf_hbm.at[pl.ds(base * L, B2 * L)], exb, gsem).wait()

    def compute(buf):
        dstb, exb, fbuf, _, _ = buf

        def grp_body(g, _):
            for j in range(L):
                eb = g * L + j
                exv = exb[pl.ds(eb * L, L)]
                for h in range(H):
                    fbuf[eb, pl.ds(h * DH, DH)] = (
                        fbuf[eb, pl.ds(h * DH, DH)] * exv[h])
            return 0
        lax.fori_loop(0, B2 // L, grp_body, 0)

    def scatter_issue(buf):
        dstb, _, fbuf, _, ssem = buf
        pltpu.async_copy(fbuf, agg_sh.at[dstb], ssem, add=True)

    def scatter_wait(buf):
        dstb, _, fbuf, _, ssem = buf
        pltpu.make_async_copy(fbuf, agg_sh.at[dstb], ssem).wait()

    def step(b, bufp, bufq):
        wait_gather(b, bufp)
        compute(bufp)

        @pl.when(b >= 1)
        def _():
            scatter_wait(bufq)

        @pl.when(b + 1 < NB2)
        def _():
            issue(b + 1, bufq)
        scatter_issue(bufp)

    issue(0, buf0)

    def batch_body(b, _):
        even = (b % 2) == 0

        @pl.when(even)
        def _():
            step(b, buf0, buf1)

        @pl.when(jnp.logical_not(even))
        def _():
            step(b, buf1, buf0)
        return 0
    lax.fori_loop(0, NB2, batch_body, 0)
    scatter_wait(buf0 if (NB2 - 1) % 2 == 0 else buf1)
    plsc.subcore_barrier()

    pltpu.sync_copy(agg_sh.at[pl.ds(s * NPT, NPT)],
                    pp_hbm.at[pl.ds(c * N_PAD + s * NPT, NPT)])


def _hop_scatter(src, dst, exf, feat):
    return pl.kernel(
        _hop_body,
        out_type=jax.ShapeDtypeStruct((2 * N_PAD, D), jnp.float32),
        mesh=_mesh2,
        scratch_types=[
            pltpu.VMEM_SHARED((N_PAD, D), jnp.float32),
            pltpu.VMEM((B2,), jnp.int32),
            pltpu.VMEM((B2,), jnp.int32),
            pltpu.VMEM((B2,), jnp.int32),
            pltpu.VMEM((B2 * L,), jnp.float32),
            pltpu.VMEM((B2 * L,), jnp.float32),
            pltpu.VMEM((B2, D), jnp.float32),
            pltpu.VMEM((B2, D), jnp.float32),
            pltpu.SemaphoreType.DMA,
            pltpu.SemaphoreType.DMA,
            pltpu.SemaphoreType.DMA,
            pltpu.SemaphoreType.DMA,
        ],
    )(src, dst, exf, feat)


def _mm_body(a_ref, w_ref, b_ref, o_ref):
    o_ref[...] = (
        jnp.dot(a_ref[...], w_ref[...], preferred_element_type=jnp.float32)
        + b_ref[...]
    )


def _mm(a, w, b):
    m, k = a.shape
    _, n = w.shape
    blk = 2000
    return pl.pallas_call(
        _mm_body,
        grid=(m // blk,),
        in_specs=[
            pl.BlockSpec((blk, k), lambda i: (i, 0)),
            pl.BlockSpec((k, n), lambda i: (0, 0)),
            pl.BlockSpec((1, n), lambda i: (0, 0)),
        ],
        out_specs=pl.BlockSpec((blk, n), lambda i: (i, 0)),
        out_shape=jax.ShapeDtypeStruct((m, n), jnp.float32),
    )(a, w, b.reshape(1, n))


def _zexp(z_ref):
    blk = z_ref.shape[0]
    z8 = z_ref[...][:, :H]
    return jnp.broadcast_to(z8[:, :, None], (blk, H, DH)).reshape(blk, D)


def _hopup_body(p0_ref, p1_ref, zinv_ref, f0_ref, o_ref):
    agg = p0_ref[...] + p1_ref[...]
    o_ref[...] = ((1.0 - ALPHA) * agg * _zexp(zinv_ref)
                  + ALPHA * f0_ref[...])


def _hop_update(p0, p1, zinv, f0):
    blk = 2000
    spec = pl.BlockSpec((blk, D), lambda i: (i, 0))
    zspec = pl.BlockSpec((blk, L), lambda i: (i, 0))
    return pl.pallas_call(
        _hopup_body,
        grid=(N // blk,),
        in_specs=[spec, spec, zspec, spec],
        out_specs=spec,
        out_shape=jax.ShapeDtypeStruct((N, D), jnp.float32),
    )(p0, p1, zinv, f0)


def _hopout_body(p0_ref, p1_ref, zinv_ref, f0_ref, wo_ref, h_ref, o_ref):
    agg = p0_ref[...] + p1_ref[...]
    feat = ((1.0 - ALPHA) * agg * _zexp(zinv_ref) + ALPHA * f0_ref[...])
    out = jnp.dot(feat, wo_ref[...],
                  preferred_element_type=jnp.float32) + h_ref[...]
    o_ref[...] = jnp.where(out > 0, out, jnp.exp(jnp.minimum(out, 0.0)) - 1.0)


def _hop_update_out(p0, p1, zinv, f0, Wo, h):
    blk = 2000
    spec = pl.BlockSpec((blk, D), lambda i: (i, 0))
    zspec = pl.BlockSpec((blk, L), lambda i: (i, 0))
    wspec = pl.BlockSpec((D, D), lambda i: (0, 0))
    return pl.pallas_call(
        _hopout_body,
        grid=(N // blk,),
        in_specs=[spec, spec, zspec, spec, wspec, spec],
        out_specs=spec,
        out_shape=jax.ShapeDtypeStruct((N, D), jnp.float32),
    )(p0, p1, zinv, f0, Wo, h)


def _layer(h, src, dst, Wq, Wk, Wv, Wo):
    Wqkv = jnp.concatenate([Wq, Wk, Wv], axis=1)
    qkv = _mm(h, Wqkv, jnp.zeros((3 * D,), jnp.float32))
    q = qkv[:, :D]
    k = qkv[:, D:2 * D]
    v = qkv[:, 2 * D:]
    qd, ks = _gather_qk(src, dst, q, k)
    exf = _edge_ex(qd, ks).reshape(E * L)
    zinv = _z_inv(dst, exf).reshape(N_PAD, L)[:N]
    feat = v
    for hop in range(HOPS):
        pp = _hop_scatter(src, dst, exf, feat)
        p0 = pp[:N]
        p1 = pp[N_PAD:N_PAD + N]
        if hop < HOPS - 1:
            feat = _hop_update(p0, p1, zinv, v)
        else:
            return _hop_update_out(p0, p1, zinv, v, Wo, h)


def kernel(x, edge_index, feat_w, feat_b, deg_emb, Wq0, Wk0, Wv0, Wo0,
           Wq1, Wk1, Wv1, Wo1, cls_w, cls_b):
    src = edge_index[0]
    dst = edge_index[1]
    h = _deg_embed(dst, deg_emb, _mm(x, feat_w, feat_b))
    h = _layer(h, src, dst, Wq0, Wk0, Wv0, Wo0)
    h = _layer(h, src, dst, Wq1, Wk1, Wv1, Wo1)
    return _mm(h, cls_w, cls_b)


# speedup vs baseline: 1.0359x; 1.0359x over previous
import math

import jax
import jax.numpy as jnp
from jax import lax
from jax.experimental import pallas as pl
from jax.experimental.pallas import tpu as pltpu
from jax.experimental.pallas import tpu_sc as plsc

N = 10000
E = 320000
D = 128
H = 8
DH = D // H
C = 40
HOPS = 4
ALPHA = 0.15
MAX_DEG = 128

NS = 16
L = 16
N_PAD = 10240
NPT = N_PAD // NS
SUB = 80
EPT1 = E // NS
EB = 2000
NB = EPT1 // EB
NW = 2 * NS
EPT2 = E // NW
B2G = 80
NBG = EPT2 // B2G
B2 = 80
NB2 = EPT2 // B2
BZ = 400
NBZ = EPT1 // BZ
ZPT = N_PAD * L // NS

_mesh1 = plsc.VectorSubcoreMesh(
    core_axis_name="c", subcore_axis_name="s", num_cores=1)
_mesh2 = plsc.VectorSubcoreMesh(
    core_axis_name="c", subcore_axis_name="s", num_cores=2)


def _deg_body(dst_hbm, emb_hbm, xw_hbm, h_hbm,
              cnt_sh, onesb, dstbuf, mycnt, degidx, embbuf, xwbuf, hbuf):
    s = lax.axis_index("s")
    zeros = jnp.zeros((L,), jnp.float32)
    ones = jnp.full((L,), 1.0, jnp.float32)

    def zb_body(i, _):
        mycnt[pl.ds(i * L, L)] = zeros
        return 0
    lax.fori_loop(0, NPT // L, zb_body, 0)
    pltpu.sync_copy(mycnt, cnt_sh.at[pl.ds(s * NPT, NPT)])

    def ob_body(i, _):
        onesb[pl.ds(i * L, L)] = ones
        return 0
    lax.fori_loop(0, EB // L, ob_body, 0)
    plsc.subcore_barrier()

    for b in range(NB):
        pltpu.sync_copy(dst_hbm.at[pl.ds(s * EPT1 + b * EB, EB)], dstbuf)
        pltpu.sync_copy(onesb, cnt_sh.at[dstbuf], add=True)
    plsc.subcore_barrier()

    pltpu.sync_copy(cnt_sh.at[pl.ds(s * NPT, NPT)], mycnt)

    def cl_body(i, _):
        degidx[i // 5, pl.ds((i % 5) * L, L)] = jnp.minimum(
            mycnt[pl.ds(i * L, L)], float(MAX_DEG - 1)).astype(jnp.int32)
        return 0
    lax.fori_loop(0, NPT // L, cl_body, 0)

    for c in range(NPT // SUB):
        node_base = s * NPT + c * SUB

        @pl.when(node_base < N)
        def _():
            pltpu.sync_copy(emb_hbm.at[degidx.at[c]], embbuf)
            pltpu.sync_copy(xw_hbm.at[pl.ds(node_base, SUB)], xwbuf)

            def add_body(i, _):
                r = i // 8
                j = (i % 8) * L
                hbuf[r, pl.ds(j, L)] = (xwbuf[r, pl.ds(j, L)]
                                        + embbuf[r, pl.ds(j, L)])
                return 0
            lax.fori_loop(0, SUB * 8, add_body, 0)
            pltpu.sync_copy(hbuf, h_hbm.at[pl.ds(node_base, SUB)])


def _deg_embed(dst, deg_emb, xw):
    return pl.kernel(
        _deg_body,
        out_type=jax.ShapeDtypeStruct((N, D), jnp.float32),
        mesh=_mesh1,
        scratch_types=[
            pltpu.VMEM_SHARED((N_PAD,), jnp.float32),
            pltpu.VMEM((EB,), jnp.float32),
            pltpu.VMEM((EB,), jnp.int32),
            pltpu.VMEM((NPT,), jnp.float32),
            pltpu.VMEM((NPT // SUB, SUB), jnp.int32),
            pltpu.VMEM((SUB, D), jnp.float32),
            pltpu.VMEM((SUB, D), jnp.float32),
            pltpu.VMEM((SUB, D), jnp.float32),
        ],
    )(dst, deg_emb, xw)


def _gather_body(src_hbm, dst_hbm, q_hbm, k_hbm, qd_hbm, ks_hbm,
                 srcall, dstall, qd0, qd1, ks0, ks1,
                 gsem0, gsem1, wsem0, wsem1):
    c = lax.axis_index("c")
    s = lax.axis_index("s")
    w = s * 2 + c
    buf0 = (qd0, ks0, gsem0, wsem0)
    buf1 = (qd1, ks1, gsem1, wsem1)

    pltpu.sync_copy(src_hbm.at[pl.ds(w * EPT2, EPT2)], srcall)
    pltpu.sync_copy(dst_hbm.at[pl.ds(w * EPT2, EPT2)], dstall)

    def issue_gather(b, buf):
        qd, ks, gsem, _ = buf
        pltpu.async_copy(q_hbm.at[dstall.at[pl.ds(b * B2G, B2G)]], qd, gsem)
        pltpu.async_copy(k_hbm.at[srcall.at[pl.ds(b * B2G, B2G)]], ks, gsem)

    def wait_gather(buf):
        qd, ks, gsem, _ = buf
        pltpu.make_async_copy(q_hbm.at[dstall.at[pl.ds(0, B2G)]], qd,
                              gsem).wait()
        pltpu.make_async_copy(k_hbm.at[srcall.at[pl.ds(0, B2G)]], ks,
                              gsem).wait()

    def issue_write(b, buf):
        qd, ks, _, wsem = buf
        base = w * EPT2 + b * B2G
        pltpu.async_copy(qd, qd_hbm.at[pl.ds(base, B2G)], wsem)
        pltpu.async_copy(ks, ks_hbm.at[pl.ds(base, B2G)], wsem)

    def wait_write(buf):
        qd, ks, _, wsem = buf
        pltpu.make_async_copy(qd, qd_hbm.at[pl.ds(0, B2G)], wsem).wait()
        pltpu.make_async_copy(ks, ks_hbm.at[pl.ds(0, B2G)], wsem).wait()

    def step(b, bufp, bufq):
        wait_gather(bufp)

        @pl.when(b >= 1)
        def _():
            wait_write(bufq)

        @pl.when(b + 1 < NBG)
        def _():
            issue_gather(b + 1, bufq)
        issue_write(b, bufp)

    issue_gather(0, buf0)

    def batch_body(b, _):
        even = (b % 2) == 0

        @pl.when(even)
        def _():
            step(b, buf0, buf1)

        @pl.when(jnp.logical_not(even))
        def _():
            step(b, buf1, buf0)
        return 0
    lax.fori_loop(0, NBG, batch_body, 0)
    wait_write(buf0 if (NBG - 1) % 2 == 0 else buf1)


def _gather_qk(src, dst, q, k):
    return pl.kernel(
        _gather_body,
        out_type=(jax.ShapeDtypeStruct((E, D), jnp.float32),
                  jax.ShapeDtypeStruct((E, D), jnp.float32)),
        mesh=_mesh2,
        scratch_types=[
            pltpu.VMEM((EPT2,), jnp.int32),
            pltpu.VMEM((EPT2,), jnp.int32),
            pltpu.VMEM((B2G, D), jnp.float32),
            pltpu.VMEM((B2G, D), jnp.float32),
            pltpu.VMEM((B2G, D), jnp.float32),
            pltpu.VMEM((B2G, D), jnp.float32),
            pltpu.SemaphoreType.DMA,
            pltpu.SemaphoreType.DMA,
            pltpu.SemaphoreType.DMA,
            pltpu.SemaphoreType.DMA,
        ],
    )(src, dst, q, k)


def _edge_body(qd_ref, ks_ref, o_ref):
    blk = qd_ref.shape[0]
    prod = qd_ref[...] * ks_ref[...]
    row = lax.broadcasted_iota(jnp.int32, (D, L), 0) // DH
    col = lax.broadcasted_iota(jnp.int32, (D, L), 1)
    sel = jnp.where(row == col, 1.0, 0.0).astype(jnp.float32)
    e16 = jnp.dot(prod, sel, preferred_element_type=jnp.float32)
    mask = (lax.broadcasted_iota(jnp.int32, (blk, L), 1) < H).astype(
        jnp.float32)
    o_ref[...] = jnp.exp(e16 * (1.0 / math.sqrt(DH))) * mask


def _edge_ex(qd, ks):
    blk = 2000
    spec = pl.BlockSpec((blk, D), lambda i: (i, 0))
    return pl.pallas_call(
        _edge_body,
        grid=(E // blk,),
        in_specs=[spec, spec],
        out_specs=pl.BlockSpec((blk, L), lambda i: (i, 0)),
        out_shape=jax.ShapeDtypeStruct((E, L), jnp.float32),
    )(qd, ks)


def _z_body(dst_hbm, exf_hbm, zinv_hbm,
            z_sh, dstb, zsrc, zidx, zbuf):
    s = lax.axis_index("s")
    iota = lax.iota(jnp.int32, L)
    zeros = jnp.zeros((L,), jnp.float32)

    def zz_body(i, _):
        zbuf[pl.ds(i * L, L)] = zeros
        return 0
    lax.fori_loop(0, ZPT // L, zz_body, 0)
    pltpu.sync_copy(zbuf, z_sh.at[pl.ds(s * ZPT, ZPT)])
    plsc.subcore_barrier()

    for b in range(NBZ):
        base = s * EPT1 + b * BZ
        pltpu.sync_copy(dst_hbm.at[pl.ds(base, BZ)], dstb)
        pltpu.sync_copy(exf_hbm.at[pl.ds(base * L, BZ * L)], zsrc)

        def grp_body(g, _):
            dv = dstb[pl.ds(g * L, L)]
            for j in range(L):
                zidx[pl.ds((g * L + j) * L, L)] = dv[j] * L + iota
            return 0
        lax.fori_loop(0, BZ // L, grp_body, 0)
        pltpu.sync_copy(zsrc, z_sh.at[zidx], add=True)
    plsc.subcore_barrier()

    pltpu.sync_copy(z_sh.at[pl.ds(s * ZPT, ZPT)], zbuf)

    def zi_body(i, _):
        zbuf[pl.ds(i * L, L)] = 1.0 / (zbuf[pl.ds(i * L, L)] + 1e-9)
        return 0
    lax.fori_loop(0, ZPT // L, zi_body, 0)
    pltpu.sync_copy(zbuf, zinv_hbm.at[pl.ds(s * ZPT, ZPT)])


def _z_inv(dst, exf):
    return pl.kernel(
        _z_body,
        out_type=jax.ShapeDtypeStruct((N_PAD * L,), jnp.float32),
        mesh=_mesh1,
        scratch_types=[
            pltpu.VMEM_SHARED((N_PAD * L,), jnp.float32),
            pltpu.VMEM((BZ,), jnp.int32),
            pltpu.VMEM((BZ * L,), jnp.float32),
            pltpu.VMEM((BZ * L,), jnp.int32),
            pltpu.VMEM((ZPT,), jnp.float32),
        ],
    )(dst, exf)


def _hop_body(src_hbm, dst_hbm, exf_hbm, feat_hbm, pp_hbm,
              agg_sh, srcb, dstb0, dstb1, exb0, exb1, fbuf0, fbuf1,
              gsem0, gsem1, ssem0, ssem1):
    c = lax.axis_index("c")
    s = lax.axis_index("s")
    w = s * 2 + c
    zeros = jnp.zeros((L,), jnp.float32)
    buf0 = (dstb0, exb0, fbuf0, gsem0, ssem0)
    buf1 = (dstb1, exb1, fbuf1, gsem1, ssem1)

    def zm_body(i, _):
        fbuf0[i // 8, pl.ds((i % 8) * L, L)] = zeros
        return 0
    lax.fori_loop(0, B2 * 8, zm_body, 0)
    for r in range(NPT // B2):
        pltpu.sync_copy(fbuf0, agg_sh.at[pl.ds(s * NPT + r * B2, B2)])
    plsc.subcore_barrier()

    def issue(bi, buf):
        dstb, exb, fbuf, gsem, _ = buf
        base = w * EPT2 + bi * B2
        pltpu.sync_copy(src_hbm.at[pl.ds(base, B2)], srcb)
        pltpu.sync_copy(dst_hbm.at[pl.ds(base, B2)], dstb)
        pltpu.async_copy(feat_hbm.at[srcb], fbuf, gsem)
        pltpu.async_copy(exf_hbm.at[pl.ds(base * L, B2 * L)], exb, gsem)

    def wait_gather(bi, buf):
        dstb, exb, fbuf, gsem, _ = buf
        base = w * EPT2 + bi * B2
        pltpu.make_async_copy(feat_hbm.at[srcb], fbuf, gsem).wait()
        pltpu.make_async_copy(
            exf_hbm.at[pl.ds(base * L, B2 * L)], exb, gsem).wait()

    def compute(buf):
        dstb, exb, fbuf, _, _ = buf

        def grp_body(g, _):
            for j in range(L):
                eb = g * L + j
                exv = exb[pl.ds(eb * L, L)]
                for h in range(H):
                    fbuf[eb, pl.ds(h * DH, DH)] = (
                        fbuf[eb, pl.ds(h * DH, DH)] * exv[h])
            return 0
        lax.fori_loop(0, B2 // L, grp_body, 0)

    def scatter_issue(buf):
        dstb, _, fbuf, _, ssem = buf
        pltpu.async_copy(fbuf, agg_sh.at[dstb], ssem, add=True)

    def scatter_wait(buf):
        dstb, _, fbuf, _, ssem = buf
        pltpu.make_async_copy(fbuf, agg_sh.at[dstb], ssem).wait()

    def step(b, bufp, bufq):
        wait_gather(b, bufp)
        compute(bufp)

        @pl.when(b >= 1)
        def _():
            scatter_wait(bufq)

        @pl.when(b + 1 < NB2)
        def _():
            issue(b + 1, bufq)
        scatter_issue(bufp)

    issue(0, buf0)

    def batch_body(b, _):
        even = (b % 2) == 0

        @pl.when(even)
        def _():
            step(b, buf0, buf1)

        @pl.when(jnp.logical_not(even))
        def _():
            step(b, buf1, buf0)
        return 0
    lax.fori_loop(0, NB2, batch_body, 0)
    scatter_wait(buf0 if (NB2 - 1) % 2 == 0 else buf1)
    plsc.subcore_barrier()

    pltpu.sync_copy(agg_sh.at[pl.ds(s * NPT, NPT)],
                    pp_hbm.at[pl.ds(c * N_PAD + s * NPT, NPT)])


def _hop_scatter(src, dst, exf, feat):
    return pl.kernel(
        _hop_body,
        out_type=jax.ShapeDtypeStruct((2 * N_PAD, D), jnp.float32),
        mesh=_mesh2,
        scratch_types=[
            pltpu.VMEM_SHARED((N_PAD, D), jnp.float32),
            pltpu.VMEM((B2,), jnp.int32),
            pltpu.VMEM((B2,), jnp.int32),
            pltpu.VMEM((B2,), jnp.int32),
            pltpu.VMEM((B2 * L,), jnp.float32),
            pltpu.VMEM((B2 * L,), jnp.float32),
            pltpu.VMEM((B2, D), jnp.float32),
            pltpu.VMEM((B2, D), jnp.float32),
            pltpu.SemaphoreType.DMA,
            pltpu.SemaphoreType.DMA,
            pltpu.SemaphoreType.DMA,
            pltpu.SemaphoreType.DMA,
        ],
    )(src, dst, exf, feat)


def _mm_body(a_ref, w_ref, b_ref, o_ref):
    o_ref[...] = (
        jnp.dot(a_ref[...], w_ref[...], preferred_element_type=jnp.float32)
        + b_ref[...]
    )


def _mm(a, w, b):
    m, k = a.shape
    _, n = w.shape
    blk = 2000
    return pl.pallas_call(
        _mm_body,
        grid=(m // blk,),
        in_specs=[
            pl.BlockSpec((blk, k), lambda i: (i, 0)),
            pl.BlockSpec((k, n), lambda i: (0, 0)),
            pl.BlockSpec((1, n), lambda i: (0, 0)),
        ],
        out_specs=pl.BlockSpec((blk, n), lambda i: (i, 0)),
        out_shape=jax.ShapeDtypeStruct((m, n), jnp.float32),
    )(a, w, b.reshape(1, n))


def _zexp(z_ref):
    blk = z_ref.shape[0]
    z8 = z_ref[...][:, :H]
    return jnp.broadcast_to(z8[:, :, None], (blk, H, DH)).reshape(blk, D)


def _hopup_body(p0_ref, p1_ref, zinv_ref, f0_ref, o_ref):
    agg = p0_ref[...] + p1_ref[...]
    o_ref[...] = ((1.0 - ALPHA) * agg * _zexp(zinv_ref)
                  + ALPHA * f0_ref[...])


def _hop_update(p0, p1, zinv, f0):
    blk = 2000
    spec = pl.BlockSpec((blk, D), lambda i: (i, 0))
    zspec = pl.BlockSpec((blk, L), lambda i: (i, 0))
    return pl.pallas_call(
        _hopup_body,
        grid=(N // blk,),
        in_specs=[spec, spec, zspec, spec],
        out_specs=spec,
        out_shape=jax.ShapeDtypeStruct((N, D), jnp.float32),
    )(p0, p1, zinv, f0)


def _hopout_body(p0_ref, p1_ref, zinv_ref, f0_ref, wo_ref, h_ref, o_ref):
    agg = p0_ref[...] + p1_ref[...]
    feat = ((1.0 - ALPHA) * agg * _zexp(zinv_ref) + ALPHA * f0_ref[...])
    out = jnp.dot(feat, wo_ref[...],
                  preferred_element_type=jnp.float32) + h_ref[...]
    o_ref[...] = jnp.where(out > 0, out, jnp.exp(jnp.minimum(out, 0.0)) - 1.0)


def _hop_update_out(p0, p1, zinv, f0, Wo, h):
    blk = 2000
    spec = pl.BlockSpec((blk, D), lambda i: (i, 0))
    zspec = pl.BlockSpec((blk, L), lambda i: (i, 0))
    wspec = pl.BlockSpec((D, D), lambda i: (0, 0))
    return pl.pallas_call(
        _hopout_body,
        grid=(N // blk,),
        in_specs=[spec, spec, zspec, spec, wspec, spec],
        out_specs=spec,
        out_shape=jax.ShapeDtypeStruct((N, D), jnp.float32),
    )(p0, p1, zinv, f0, Wo, h)


def _layer(h, src, dst, Wq, Wk, Wv, Wo):
    Wqkv = jnp.concatenate([Wq, Wk, Wv], axis=1)
    qkv = _mm(h, Wqkv, jnp.zeros((3 * D,), jnp.float32))
    q = qkv[:, :D]
    k = qkv[:, D:2 * D]
    v = qkv[:, 2 * D:]
    qd, ks = _gather_qk(src, dst, q, k)
    exf = _edge_ex(qd, ks).reshape(E * L)
    zinv = _z_inv(dst, exf).reshape(N_PAD, L)[:N]
    feat = v
    for hop in range(HOPS):
        pp = _hop_scatter(src, dst, exf, feat)
        p0 = pp[:N]
        p1 = pp[N_PAD:N_PAD + N]
        if hop < HOPS - 1:
            feat = _hop_update(p0, p1, zinv, v)
        else:
            return _hop_update_out(p0, p1, zinv, v, Wo, h)


def kernel(x, edge_index, feat_w, feat_b, deg_emb, Wq0, Wk0, Wv0, Wo0,
           Wq1, Wk1, Wv1, Wo1, cls_w, cls_b):
    src = edge_index[0]
    dst = edge_index[1]
    h = _deg_embed(dst, deg_emb, _mm(x, feat_w, feat_b))
    h = _layer(h, src, dst, Wq0, Wk0, Wv0, Wo0)
    h = _layer(h, src, dst, Wq1, Wk1, Wv1, Wo1)
    return _mm(h, cls_w, cls_b)


# speedup vs baseline: 1.1291x; 1.0899x over previous
import math

import jax
import jax.numpy as jnp
from jax import lax
from jax.experimental import pallas as pl
from jax.experimental.pallas import tpu as pltpu
from jax.experimental.pallas import tpu_sc as plsc

N = 10000
E = 320000
D = 128
H = 8
DH = D // H
C = 40
HOPS = 4
ALPHA = 0.15
MAX_DEG = 128

NS = 16
L = 16
N_PAD = 10240
NPT = N_PAD // NS
SUB = 80
EPT1 = E // NS
EB = 2000
NB = EPT1 // EB
NW = 2 * NS
EPT2 = E // NW
B2G = 80
NBG = EPT2 // B2G
B2 = 80
NB2 = EPT2 // B2
BZ = 400
NBZ = EPT1 // BZ
ZPT = N_PAD * L // NS

_mesh1 = plsc.VectorSubcoreMesh(
    core_axis_name="c", subcore_axis_name="s", num_cores=1)
_mesh2 = plsc.VectorSubcoreMesh(
    core_axis_name="c", subcore_axis_name="s", num_cores=2)


def _deg_body(dst_hbm, emb_hbm, xw_hbm, h_hbm,
              cnt_sh, onesb, dstbuf, mycnt, degidx, embbuf, xwbuf, hbuf):
    s = lax.axis_index("s")
    zeros = jnp.zeros((L,), jnp.float32)
    ones = jnp.full((L,), 1.0, jnp.float32)

    def zb_body(i, _):
        mycnt[pl.ds(i * L, L)] = zeros
        return 0
    lax.fori_loop(0, NPT // L, zb_body, 0)
    pltpu.sync_copy(mycnt, cnt_sh.at[pl.ds(s * NPT, NPT)])

    def ob_body(i, _):
        onesb[pl.ds(i * L, L)] = ones
        return 0
    lax.fori_loop(0, EB // L, ob_body, 0)
    plsc.subcore_barrier()

    for b in range(NB):
        pltpu.sync_copy(dst_hbm.at[pl.ds(s * EPT1 + b * EB, EB)], dstbuf)
        pltpu.sync_copy(onesb, cnt_sh.at[dstbuf], add=True)
    plsc.subcore_barrier()

    pltpu.sync_copy(cnt_sh.at[pl.ds(s * NPT, NPT)], mycnt)

    def cl_body(i, _):
        degidx[i // 5, pl.ds((i % 5) * L, L)] = jnp.minimum(
            mycnt[pl.ds(i * L, L)], float(MAX_DEG - 1)).astype(jnp.int32)
        return 0
    lax.fori_loop(0, NPT // L, cl_body, 0)

    for c in range(NPT // SUB):
        node_base = s * NPT + c * SUB

        @pl.when(node_base < N)
        def _():
            pltpu.sync_copy(emb_hbm.at[degidx.at[c]], embbuf)
            pltpu.sync_copy(xw_hbm.at[pl.ds(node_base, SUB)], xwbuf)

            def add_body(i, _):
                r = i // 8
                j = (i % 8) * L
                hbuf[r, pl.ds(j, L)] = (xwbuf[r, pl.ds(j, L)]
                                        + embbuf[r, pl.ds(j, L)])
                return 0
            lax.fori_loop(0, SUB * 8, add_body, 0)
            pltpu.sync_copy(hbuf, h_hbm.at[pl.ds(node_base, SUB)])


def _deg_embed(dst, deg_emb, xw):
    return pl.kernel(
        _deg_body,
        out_type=jax.ShapeDtypeStruct((N, D), jnp.float32),
        mesh=_mesh1,
        scratch_types=[
            pltpu.VMEM_SHARED((N_PAD,), jnp.float32),
            pltpu.VMEM((EB,), jnp.float32),
            pltpu.VMEM((EB,), jnp.int32),
            pltpu.VMEM((NPT,), jnp.float32),
            pltpu.VMEM((NPT // SUB, SUB), jnp.int32),
            pltpu.VMEM((SUB, D), jnp.float32),
            pltpu.VMEM((SUB, D), jnp.float32),
            pltpu.VMEM((SUB, D), jnp.float32),
        ],
    )(dst, deg_emb, xw)


def _gather_body(src_hbm, dst_hbm, q_hbm, k_hbm, qd_hbm, ks_hbm,
                 srcall, dstall, qd0, qd1, ks0, ks1,
                 gsem0, gsem1, wsem0, wsem1):
    c = lax.axis_index("c")
    s = lax.axis_index("s")
    w = s * 2 + c
    buf0 = (qd0, ks0, gsem0, wsem0)
    buf1 = (qd1, ks1, gsem1, wsem1)

    pltpu.sync_copy(src_hbm.at[pl.ds(w * EPT2, EPT2)], srcall)
    pltpu.sync_copy(dst_hbm.at[pl.ds(w * EPT2, EPT2)], dstall)

    def issue_gather(b, buf):
        qd, ks, gsem, _ = buf
        pltpu.async_copy(q_hbm.at[dstall.at[pl.ds(b * B2G, B2G)]], qd, gsem)
        pltpu.async_copy(k_hbm.at[srcall.at[pl.ds(b * B2G, B2G)]], ks, gsem)

    def wait_gather(buf):
        qd, ks, gsem, _ = buf
        pltpu.make_async_copy(q_hbm.at[dstall.at[pl.ds(0, B2G)]], qd,
                              gsem).wait()
        pltpu.make_async_copy(k_hbm.at[srcall.at[pl.ds(0, B2G)]], ks,
                              gsem).wait()

    def issue_write(b, buf):
        qd, ks, _, wsem = buf
        base = w * EPT2 + b * B2G
        pltpu.async_copy(qd, qd_hbm.at[pl.ds(base, B2G)], wsem)
        pltpu.async_copy(ks, ks_hbm.at[pl.ds(base, B2G)], wsem)

    def wait_write(buf):
        qd, ks, _, wsem = buf
        pltpu.make_async_copy(qd, qd_hbm.at[pl.ds(0, B2G)], wsem).wait()
        pltpu.make_async_copy(ks, ks_hbm.at[pl.ds(0, B2G)], wsem).wait()

    def step(b, bufp, bufq):
        wait_gather(bufp)

        @pl.when(b >= 1)
        def _():
            wait_write(bufq)

        @pl.when(b + 1 < NBG)
        def _():
            issue_gather(b + 1, bufq)
        issue_write(b, bufp)

    issue_gather(0, buf0)

    def batch_body(b, _):
        even = (b % 2) == 0

        @pl.when(even)
        def _():
            step(b, buf0, buf1)

        @pl.when(jnp.logical_not(even))
        def _():
            step(b, buf1, buf0)
        return 0
    lax.fori_loop(0, NBG, batch_body, 0)
    wait_write(buf0 if (NBG - 1) % 2 == 0 else buf1)


def _gather_qk(src, dst, q, k):
    return pl.kernel(
        _gather_body,
        out_type=(jax.ShapeDtypeStruct((E, D), jnp.float32),
                  jax.ShapeDtypeStruct((E, D), jnp.float32)),
        mesh=_mesh2,
        scratch_types=[
            pltpu.VMEM((EPT2,), jnp.int32),
            pltpu.VMEM((EPT2,), jnp.int32),
            pltpu.VMEM((B2G, D), jnp.float32),
            pltpu.VMEM((B2G, D), jnp.float32),
            pltpu.VMEM((B2G, D), jnp.float32),
            pltpu.VMEM((B2G, D), jnp.float32),
            pltpu.SemaphoreType.DMA,
            pltpu.SemaphoreType.DMA,
            pltpu.SemaphoreType.DMA,
            pltpu.SemaphoreType.DMA,
        ],
    )(src, dst, q, k)


def _edge_body(qd_ref, ks_ref, o_ref):
    blk = qd_ref.shape[0]
    prod = qd_ref[...] * ks_ref[...]
    row = lax.broadcasted_iota(jnp.int32, (D, L), 0) // DH
    col = lax.broadcasted_iota(jnp.int32, (D, L), 1)
    sel = jnp.where(row == col, 1.0, 0.0).astype(jnp.float32)
    e16 = jnp.dot(prod, sel, preferred_element_type=jnp.float32)
    mask = (lax.broadcasted_iota(jnp.int32, (blk, L), 1) < H).astype(
        jnp.float32)
    o_ref[...] = jnp.exp(e16 * (1.0 / math.sqrt(DH))) * mask


def _edge_ex(qd, ks):
    blk = 2000
    spec = pl.BlockSpec((blk, D), lambda i: (i, 0))
    return pl.pallas_call(
        _edge_body,
        grid=(E // blk,),
        in_specs=[spec, spec],
        out_specs=pl.BlockSpec((blk, L), lambda i: (i, 0)),
        out_shape=jax.ShapeDtypeStruct((E, L), jnp.float32),
    )(qd, ks)


def _z_body(dst_hbm, exf_hbm, zinv_hbm,
            z_sh, dstb, zsrc, zidx, zbuf):
    s = lax.axis_index("s")
    iota = lax.iota(jnp.int32, L)
    zeros = jnp.zeros((L,), jnp.float32)

    def zz_body(i, _):
        zbuf[pl.ds(i * L, L)] = zeros
        return 0
    lax.fori_loop(0, ZPT // L, zz_body, 0)
    pltpu.sync_copy(zbuf, z_sh.at[pl.ds(s * ZPT, ZPT)])
    plsc.subcore_barrier()

    for b in range(NBZ):
        base = s * EPT1 + b * BZ
        pltpu.sync_copy(dst_hbm.at[pl.ds(base, BZ)], dstb)
        pltpu.sync_copy(exf_hbm.at[pl.ds(base * L, BZ * L)], zsrc)

        def grp_body(g, _):
            dv = dstb[pl.ds(g * L, L)]
            for j in range(L):
                zidx[pl.ds((g * L + j) * L, L)] = dv[j] * L + iota
            return 0
        lax.fori_loop(0, BZ // L, grp_body, 0)
        pltpu.sync_copy(zsrc, z_sh.at[zidx], add=True)
    plsc.subcore_barrier()

    pltpu.sync_copy(z_sh.at[pl.ds(s * ZPT, ZPT)], zbuf)

    def zi_body(i, _):
        zbuf[pl.ds(i * L, L)] = 1.0 / (zbuf[pl.ds(i * L, L)] + 1e-9)
        return 0
    lax.fori_loop(0, ZPT // L, zi_body, 0)
    pltpu.sync_copy(zbuf, zinv_hbm.at[pl.ds(s * ZPT, ZPT)])


def _z_inv(dst, exf):
    return pl.kernel(
        _z_body,
        out_type=jax.ShapeDtypeStruct((N_PAD * L,), jnp.float32),
        mesh=_mesh1,
        scratch_types=[
            pltpu.VMEM_SHARED((N_PAD * L,), jnp.float32),
            pltpu.VMEM((BZ,), jnp.int32),
            pltpu.VMEM((BZ * L,), jnp.float32),
            pltpu.VMEM((BZ * L,), jnp.int32),
            pltpu.VMEM((ZPT,), jnp.float32),
        ],
    )(dst, exf)


def _hop_body(src_hbm, dst_hbm, exf_hbm, feat_hbm, pp_hbm,
              agg_sh, srcall, dstb0, dstb1, exb0, exb1, fbuf0, fbuf1,
              gsem0, gsem1, ssem0, ssem1):
    c = lax.axis_index("c")
    s = lax.axis_index("s")
    w = s * 2 + c
    zeros = jnp.zeros((L,), jnp.float32)
    buf0 = (dstb0, exb0, fbuf0, gsem0, ssem0)
    buf1 = (dstb1, exb1, fbuf1, gsem1, ssem1)

    def zm_body(i, _):
        fbuf0[i // 8, pl.ds((i % 8) * L, L)] = zeros
        return 0
    lax.fori_loop(0, B2 * 8, zm_body, 0)
    for r in range(NPT // B2):
        pltpu.sync_copy(fbuf0, agg_sh.at[pl.ds(s * NPT + r * B2, B2)])
    pltpu.sync_copy(src_hbm.at[pl.ds(w * EPT2, EPT2)], srcall)
    plsc.subcore_barrier()

    def issue(bi, buf):
        dstb, exb, fbuf, gsem, _ = buf
        base = w * EPT2 + bi * B2
        pltpu.sync_copy(dst_hbm.at[pl.ds(base, B2)], dstb)
        pltpu.async_copy(feat_hbm.at[srcall.at[pl.ds(bi * B2, B2)]],
                         fbuf, gsem)
        pltpu.async_copy(exf_hbm.at[pl.ds(base * L, B2 * L)], exb, gsem)

    def wait_gather(bi, buf):
        dstb, exb, fbuf, gsem, _ = buf
        base = w * EPT2 + bi * B2
        pltpu.make_async_copy(feat_hbm.at[srcall.at[pl.ds(0, B2)]],
                              fbuf, gsem).wait()
        pltpu.make_async_copy(
            exf_hbm.at[pl.ds(base * L, B2 * L)], exb, gsem).wait()

    def compute(buf):
        dstb, exb, fbuf, _, _ = buf

        def grp_body(g, _):
            for j in range(L):
                eb = g * L + j
                exv = exb[pl.ds(eb * L, L)]
                for h in range(H):
                    fbuf[eb, pl.ds(h * DH, DH)] = (
                        fbuf[eb, pl.ds(h * DH, DH)] * exv[h])
            return 0
        lax.fori_loop(0, B2 // L, grp_body, 0)

    def scatter_issue(buf):
        dstb, _, fbuf, _, ssem = buf
        pltpu.async_copy(fbuf, agg_sh.at[dstb], ssem, add=True)

    def scatter_wait(buf):
        dstb, _, fbuf, _, ssem = buf
        pltpu.make_async_copy(fbuf, agg_sh.at[dstb], ssem).wait()

    def step(b, bufp, bufq):
        wait_gather(b, bufp)
        compute(bufp)

        @pl.when(b >= 1)
        def _():
            scatter_wait(bufq)

        @pl.when(b + 1 < NB2)
        def _():
            issue(b + 1, bufq)
        scatter_issue(bufp)

    issue(0, buf0)

    def batch_body(b, _):
        even = (b % 2) == 0

        @pl.when(even)
        def _():
            step(b, buf0, buf1)

        @pl.when(jnp.logical_not(even))
        def _():
            step(b, buf1, buf0)
        return 0
    lax.fori_loop(0, NB2, batch_body, 0)
    scatter_wait(buf0 if (NB2 - 1) % 2 == 0 else buf1)
    plsc.subcore_barrier()

    pltpu.sync_copy(agg_sh.at[pl.ds(s * NPT, NPT)],
                    pp_hbm.at[pl.ds(c * N_PAD + s * NPT, NPT)])


def _hop_scatter(src, dst, exf, feat):
    return pl.kernel(
        _hop_body,
        out_type=jax.ShapeDtypeStruct((2 * N_PAD, D), jnp.float32),
        mesh=_mesh2,
        scratch_types=[
            pltpu.VMEM_SHARED((N_PAD, D), jnp.float32),
            pltpu.VMEM((EPT2,), jnp.int32),
            pltpu.VMEM((B2,), jnp.int32),
            pltpu.VMEM((B2,), jnp.int32),
            pltpu.VMEM((B2 * L,), jnp.float32),
            pltpu.VMEM((B2 * L,), jnp.float32),
            pltpu.VMEM((B2, D), jnp.float32),
            pltpu.VMEM((B2, D), jnp.float32),
            pltpu.SemaphoreType.DMA,
            pltpu.SemaphoreType.DMA,
            pltpu.SemaphoreType.DMA,
            pltpu.SemaphoreType.DMA,
        ],
    )(src, dst, exf, feat)


def _mm_body(a_ref, w_ref, b_ref, o_ref):
    o_ref[...] = (
        jnp.dot(a_ref[...], w_ref[...], preferred_element_type=jnp.float32)
        + b_ref[...]
    )


def _mm(a, w, b):
    m, k = a.shape
    _, n = w.shape
    blk = 2000
    return pl.pallas_call(
        _mm_body,
        grid=(m // blk,),
        in_specs=[
            pl.BlockSpec((blk, k), lambda i: (i, 0)),
            pl.BlockSpec((k, n), lambda i: (0, 0)),
            pl.BlockSpec((1, n), lambda i: (0, 0)),
        ],
        out_specs=pl.BlockSpec((blk, n), lambda i: (i, 0)),
        out_shape=jax.ShapeDtypeStruct((m, n), jnp.float32),
    )(a, w, b.reshape(1, n))


def _zexp(z_ref):
    blk = z_ref.shape[0]
    z8 = z_ref[...][:, :H]
    return jnp.broadcast_to(z8[:, :, None], (blk, H, DH)).reshape(blk, D)


def _hopup_body(p0_ref, p1_ref, zinv_ref, f0_ref, o_ref):
    agg = p0_ref[...] + p1_ref[...]
    o_ref[...] = ((1.0 - ALPHA) * agg * _zexp(zinv_ref)
                  + ALPHA * f0_ref[...])


def _hop_update(p0, p1, zinv, f0):
    blk = 2000
    spec = pl.BlockSpec((blk, D), lambda i: (i, 0))
    zspec = pl.BlockSpec((blk, L), lambda i: (i, 0))
    return pl.pallas_call(
        _hopup_body,
        grid=(N // blk,),
        in_specs=[spec, spec, zspec, spec],
        out_specs=spec,
        out_shape=jax.ShapeDtypeStruct((N, D), jnp.float32),
    )(p0, p1, zinv, f0)


def _hopout_body(p0_ref, p1_ref, zinv_ref, f0_ref, wo_ref, h_ref, o_ref):
    agg = p0_ref[...] + p1_ref[...]
    feat = ((1.0 - ALPHA) * agg * _zexp(zinv_ref) + ALPHA * f0_ref[...])
    out = jnp.dot(feat, wo_ref[...],
                  preferred_element_type=jnp.float32) + h_ref[...]
    o_ref[...] = jnp.where(out > 0, out, jnp.exp(jnp.minimum(out, 0.0)) - 1.0)


def _hop_update_out(p0, p1, zinv, f0, Wo, h):
    blk = 2000
    spec = pl.BlockSpec((blk, D), lambda i: (i, 0))
    zspec = pl.BlockSpec((blk, L), lambda i: (i, 0))
    wspec = pl.BlockSpec((D, D), lambda i: (0, 0))
    return pl.pallas_call(
        _hopout_body,
        grid=(N // blk,),
        in_specs=[spec, spec, zspec, spec, wspec, spec],
        out_specs=spec,
        out_shape=jax.ShapeDtypeStruct((N, D), jnp.float32),
    )(p0, p1, zinv, f0, Wo, h)


def _layer(h, src, dst, Wq, Wk, Wv, Wo):
    Wqkv = jnp.concatenate([Wq, Wk, Wv], axis=1)
    qkv = _mm(h, Wqkv, jnp.zeros((3 * D,), jnp.float32))
    q = qkv[:, :D]
    k = qkv[:, D:2 * D]
    v = qkv[:, 2 * D:]
    qd, ks = _gather_qk(src, dst, q, k)
    exf = _edge_ex(qd, ks).reshape(E * L)
    zinv = _z_inv(dst, exf).reshape(N_PAD, L)[:N]
    feat = v
    for hop in range(HOPS):
        pp = _hop_scatter(src, dst, exf, feat)
        p0 = pp[:N]
        p1 = pp[N_PAD:N_PAD + N]
        if hop < HOPS - 1:
            feat = _hop_update(p0, p1, zinv, v)
        else:
            return _hop_update_out(p0, p1, zinv, v, Wo, h)


def kernel(x, edge_index, feat_w, feat_b, deg_emb, Wq0, Wk0, Wv0, Wo0,
           Wq1, Wk1, Wv1, Wo1, cls_w, cls_b):
    src = edge_index[0]
    dst = edge_index[1]
    h = _deg_embed(dst, deg_emb, _mm(x, feat_w, feat_b))
    h = _layer(h, src, dst, Wq0, Wk0, Wv0, Wo0)
    h = _layer(h, src, dst, Wq1, Wk1, Wv1, Wo1)
    return _mm(h, cls_w, cls_b)
